# baseline (speedup 1.0000x reference)
import functools
import math

import numpy as np
import jax
import jax.numpy as jnp
from jax import lax
from jax.experimental import pallas as pl
from jax.experimental.pallas import tpu as pltpu
from jax.experimental.pallas import tpu_sc as plsc

_NUM_FIELDS = 26
_NUM_GROUPS = 4
_BASE_DIM = 128
_BATCH = 16384
_FIELD_DIM = 100000


def _group_structure():
    np.random.seed(0)
    dim_indices = np.arange(_NUM_FIELDS)
    np.random.shuffle(dim_indices)
    chunk = _NUM_FIELDS // _NUM_GROUPS
    groups = []
    for i in range(_NUM_GROUPS):
        if i == _NUM_GROUPS - 1:
            groups.append(dim_indices[i * chunk:])
            break
        groups.append(dim_indices[i * chunk:(i + 1) * chunk])
    total = _NUM_FIELDS * _FIELD_DIM
    emb_dims = []
    for g in groups:
        div = total / (len(g) * _FIELD_DIM)
        emb_dims.append(max(2, int(_BASE_DIM / 2 ** int(math.log2(div)))))
    return groups, emb_dims


_GROUPS, _EMB_DIMS = _group_structure()
_NFIELDS = [len(g) for g in _GROUPS]
_FLAT_COLS = np.concatenate(_GROUPS).astype(np.int32)
_FLAT_OFFS = np.concatenate(
    [np.arange(n, dtype=np.int64) * _FIELD_DIM for n in _NFIELDS]).astype(np.int32)
_GROUP_ROW0 = np.cumsum([0] + _NFIELDS)[:4]
_POOL_COL0 = np.cumsum([0] + _EMB_DIMS)[:4]
_POOL_DIM = int(sum(_EMB_DIMS))

_NC, _NS, _L = 2, 16, 16
_NW = _NC * _NS
_BPW = _BATCH // _NW
_CHUNK = 128
_NCHUNK = _BPW // _CHUNK


def _sc_body(idx_flat, e0, e1, e2, e3, out0, out1, out2, out3,
             idx_s, g32, a32, g64, a64, sem):
    tables = [e0, e1, e2, e3]
    outs = [out0, out1, out2, out3]
    wid = lax.axis_index("s") * _NC + lax.axis_index("c")

    def chunk_body(ch, carry):
        base = wid * _BPW + ch * _CHUNK
        for g in range(_NUM_GROUPS):
            d = _EMB_DIMS[g]
            n = _NFIELDS[g]
            acc = a32 if d == 32 else a64
            gbuf = g32 if d == 32 else g64
            table = tables[g]
            for j in range(n):
                row = int(_GROUP_ROW0[g]) + j
                off = int(_FLAT_OFFS[row])
                pltpu.sync_copy(idx_flat.at[pl.ds(row * _BATCH + base, _CHUNK)],
                                idx_s)
                if off != 0:
                    for i in range(_CHUNK // _L):
                        sl = pl.ds(i * _L, _L)
                        idx_s[sl] = idx_s[sl] + off
                dst = acc if j == 0 else gbuf
                pltpu.async_copy(table.at[idx_s], dst, sem).wait()
                if j > 0:
                    nb = d // _L

                    def add_body(i, c):
                        for k2 in range(nb):
                            sl = pl.ds(k2 * _L, _L)
                            plsc.addupdate(acc.at[i, sl], gbuf[i, sl])
                        return c

                    lax.fori_loop(0, _CHUNK, add_body, 0, unroll=8)
            pltpu.sync_copy(acc, outs[g].at[pl.ds(base, _CHUNK), :])
        return carry

    lax.fori_loop(0, _NCHUNK, chunk_body, 0)


def _embedding_bag_pooled(idx_flat, e0, e1, e2, e3):
    mesh = plsc.VectorSubcoreMesh(
        core_axis_name="c", subcore_axis_name="s",
        num_cores=_NC, num_subcores=_NS)
    kern = pl.kernel(
        _sc_body,
        out_type=tuple(
            jax.ShapeDtypeStruct((_BATCH, d), jnp.float32) for d in _EMB_DIMS),
        mesh=mesh,
        scratch_types=[
            pltpu.VMEM((_CHUNK,), jnp.int32),
            pltpu.VMEM((_CHUNK, 32), jnp.float32),
            pltpu.VMEM((_CHUNK, 32), jnp.float32),
            pltpu.VMEM((_CHUNK, 64), jnp.float32),
            pltpu.VMEM((_CHUNK, 64), jnp.float32),
            pltpu.SemaphoreType.DMA,
        ],
        compiler_params=pltpu.CompilerParams(use_tc_tiling_on_sc=False),
    )
    return kern(idx_flat, e0, e1, e2, e3)


def _mm_body(x0_ref, x1_ref, x2_ref, x3_ref, w_ref, o_ref):
    x = jnp.concatenate(
        [x0_ref[...], x1_ref[...], x2_ref[...], x3_ref[...]], axis=1)
    o_ref[...] = jnp.dot(x, w_ref[...], preferred_element_type=jnp.float32)


def _project(pooled, w):
    blk = 2048
    return pl.pallas_call(
        _mm_body,
        grid=(_BATCH // blk,),
        in_specs=[
            pl.BlockSpec((blk, d), lambda i: (i, 0)) for d in _EMB_DIMS
        ] + [
            pl.BlockSpec((_POOL_DIM, _BASE_DIM), lambda i: (0, 0)),
        ],
        out_specs=pl.BlockSpec((blk, _BASE_DIM), lambda i: (i, 0)),
        out_shape=jax.ShapeDtypeStruct((_BATCH, _BASE_DIM), jnp.float32),
    )(*pooled, w)


def kernel(input_, embed_w_0, linear_w_0, embed_w_1, linear_w_1,
           embed_w_2, linear_w_2, embed_w_3, linear_w_3):
    idx_flat = jnp.transpose(input_[:, _FLAT_COLS]).reshape(-1)
    pooled = _embedding_bag_pooled(idx_flat, embed_w_0, embed_w_1,
                                   embed_w_2, embed_w_3)
    w = jnp.concatenate([linear_w_0.T, linear_w_1.T,
                         linear_w_2.T, linear_w_3.T], axis=0)
    return _project(pooled, w)
